# baseline (speedup 1.0000x reference)
import functools

import jax
import jax.numpy as jnp
from jax import lax
from jax.experimental import pallas as pl
from jax.experimental.pallas import tpu as pltpu, tpu_sc as plsc

B = 1024
N = 100000
NUM_WORKERS = 32
ROWS_PER_WORKER = B // NUM_WORKERS
CHUNK = 20000
CHUNKS_PER_ROW = N // CHUNK
CHUNKS_PER_WORKER = ROWS_PER_WORKER * CHUNKS_PER_ROW
ITERS_PER_CHUNK = CHUNK // 16
UNROLL = 5
LANES = 16
BIG_I32 = 0x7FFFFFFF


def _body(pred_hbm, tgt_hbm, out_hbm, buf0, buf1, tbuf, obuf, sem0, sem1):
    wid = lax.axis_index("s") * 2 + lax.axis_index("c")
    wbase = wid * (ROWS_PER_WORKER * N)
    bufs = (buf0, buf1)
    sems = (sem0, sem1)

    pltpu.sync_copy(tgt_hbm.at[pl.ds(wid * ROWS_PER_WORKER, ROWS_PER_WORKER)],
                    tbuf)

    lane = lax.iota(jnp.int32, LANES)
    minus_inf = jnp.full((LANES,), -jnp.inf, jnp.float32)
    zeros_i = jnp.zeros((LANES,), jnp.int32)

    def src(c):
        return pred_hbm.at[pl.ds(wbase + c * CHUNK, CHUNK)]

    pltpu.async_copy(src(0), buf0, sem0)

    def chunk_step(b, c, state):
        m, vidx, argvec, cvec = state
        nxt = jnp.minimum(c + 1, CHUNKS_PER_WORKER - 1)
        pltpu.async_copy(src(nxt), bufs[1 - b], sems[1 - b])
        pltpu.make_async_copy(src(c), bufs[b], sems[b]).wait()

        p = c % CHUNKS_PER_ROW
        r = c // CHUNKS_PER_ROW
        row_start = p == 0
        m = jnp.where(row_start, minus_inf, m)
        vidx = jnp.where(row_start, zeros_i, vidx)

        col0 = p * CHUNK
        buf = bufs[b]

        def inner(k, st):
            m, vidx = st
            for u in range(UNROLL):
                kk = k * UNROLL + u
                v = buf[pl.ds(kk * 16, 16)]
                base = jnp.full((LANES,), col0 + kk * 16, jnp.int32)
                nm = jnp.maximum(m, v)
                vidx = jnp.where(nm != m, base, vidx)
                m = nm
            return m, vidx

        m, vidx = lax.fori_loop(0, ITERS_PER_CHUNK // UNROLL, inner,
                                (m, vidx), unroll=False)

        row_end = p == CHUNKS_PER_ROW - 1
        cols = vidx + lane
        rm = jnp.float32(-jnp.inf)
        ai = jnp.int32(BIG_I32)
        for i in range(LANES):
            vi = m[i]
            ci = cols[i]
            better = jnp.logical_or(
                vi > rm, jnp.logical_and(vi == rm, ci < ai))
            rm = jnp.where(better, vi, rm)
            ai = jnp.where(better, ci, ai)
        jslot = r % LANES
        slot_sel = jnp.where(row_end, jslot, jnp.int32(-1))
        upd = lane == jnp.full((LANES,), slot_sel)
        argvec = jnp.where(upd, jnp.full((LANES,), ai), argvec)

        grp_end = jnp.logical_and(row_end, jslot == LANES - 1)
        tg = tbuf[pl.ds((r // LANES) * LANES, LANES)]
        inc = jnp.where(argvec == tg, jnp.int32(1), jnp.int32(0))
        cvec = jnp.where(grp_end, cvec + inc, cvec)
        return m, vidx, argvec, cvec

    def outer(c2, state):
        for b in range(2):
            state = chunk_step(b, c2 * 2 + b, state)
        return state

    init = (minus_inf, zeros_i, zeros_i, zeros_i)
    _, _, _, cvec = lax.fori_loop(0, CHUNKS_PER_WORKER // 2, outer, init,
                                  unroll=False)

    pltpu.make_async_copy(src(CHUNKS_PER_WORKER - 1), buf0, sem0).wait()

    obuf[...] = cvec.astype(jnp.float32) * jnp.float32(100.0 / B)
    pltpu.sync_copy(obuf, out_hbm.at[pl.ds(wid * LANES, LANES)])


@jax.jit
def _bbox_top1(pred_flat, tgt):
    mesh = plsc.VectorSubcoreMesh(core_axis_name="c", subcore_axis_name="s")
    partials = pl.kernel(
        _body,
        out_type=jax.ShapeDtypeStruct((NUM_WORKERS * LANES,), jnp.float32),
        mesh=mesh,
        scratch_types=[
            pltpu.VMEM((CHUNK,), jnp.float32),
            pltpu.VMEM((CHUNK,), jnp.float32),
            pltpu.VMEM((ROWS_PER_WORKER,), jnp.int32),
            pltpu.VMEM((LANES,), jnp.float32),
            pltpu.SemaphoreType.DMA,
            pltpu.SemaphoreType.DMA,
        ],
    )(pred_flat, tgt)
    return jnp.sum(partials).reshape(1)


def kernel(pred, target):
    pred_flat = pred.reshape(-1)
    tgt = target.astype(jnp.int32)
    return _bbox_top1(pred_flat, tgt)
